# baseline (speedup 1.0000x reference)
import functools

import jax
import jax.numpy as jnp
from jax import lax
from jax.experimental import pallas as pl
from jax.experimental.pallas import tpu as pltpu
from jax.experimental.pallas import tpu_sc as plsc

_MAX_LEN = 2048
_DIM = 1024
_NC = 2
_ROWS_PER_C = _MAX_LEN // _NC
_N_CHUNK = 8
_CH = _ROWS_PER_C // _N_CHUNK

_mesh = plsc.ScalarSubcoreMesh(axis_name="c")


@functools.partial(
    pl.kernel,
    mesh=_mesh,
    out_type=jax.ShapeDtypeStruct((_MAX_LEN, _DIM), jnp.float32),
    scratch_types=[
        pltpu.VMEM_SHARED((_ROWS_PER_C, _DIM), jnp.float32),
        pltpu.SemaphoreType.DMA,
        pltpu.SemaphoreType.DMA,
    ],
)
def _pos_embed_copy(w_hbm, out_hbm, sbuf, sem_in, sem_out):
    base = lax.axis_index("c") * _ROWS_PER_C

    ins = [
        pltpu.make_async_copy(
            w_hbm.at[pl.ds(base + i * _CH, _CH)],
            sbuf.at[pl.ds(i * _CH, _CH)],
            sem_in,
        )
        for i in range(_N_CHUNK)
    ]
    outs = [
        pltpu.make_async_copy(
            sbuf.at[pl.ds(i * _CH, _CH)],
            out_hbm.at[pl.ds(base + i * _CH, _CH)],
            sem_out,
        )
        for i in range(_N_CHUNK)
    ]
    for c in ins:
        c.start()
    for i in range(_N_CHUNK):
        ins[i].wait()
        outs[i].start()
    for c in outs:
        c.wait()


def kernel(x, W):
    del x
    return _pos_embed_copy(W)


# speedup vs baseline: 1.0334x; 1.0334x over previous
import functools

import jax
import jax.numpy as jnp
from jax import lax
from jax.experimental import pallas as pl
from jax.experimental.pallas import tpu as pltpu
from jax.experimental.pallas import tpu_sc as plsc

_MAX_LEN = 2048
_DIM = 1024
_NC = 2
_NS = 16
_NW = _NC * _NS
_ROWS_PER_W = _MAX_LEN // _NW
_TS_ROWS = 48
_SP_ROWS = _ROWS_PER_W - _TS_ROWS

_mesh = plsc.VectorSubcoreMesh(core_axis_name="c", subcore_axis_name="s")


@functools.partial(
    pl.kernel,
    mesh=_mesh,
    out_type=jax.ShapeDtypeStruct((_MAX_LEN, _DIM), jnp.float32),
    scratch_types=[
        pltpu.VMEM((_TS_ROWS, _DIM), jnp.float32),
        pltpu.VMEM_SHARED((_NS, _SP_ROWS, _DIM), jnp.float32),
        pltpu.SemaphoreType.DMA,
        pltpu.SemaphoreType.DMA,
        pltpu.SemaphoreType.DMA,
        pltpu.SemaphoreType.DMA,
    ],
)
def _pos_embed_copy(w_hbm, out_hbm, tbuf, sbuf, sem_ti, sem_to, sem_si, sem_so):
    sid = lax.axis_index("s")
    wid = sid * _NC + lax.axis_index("c")
    base = wid * _ROWS_PER_W

    ts_in = pltpu.make_async_copy(w_hbm.at[pl.ds(base, _TS_ROWS)], tbuf, sem_ti)
    ts_in.start()
    sp_in = pltpu.make_async_copy(
        w_hbm.at[pl.ds(base + _TS_ROWS, _SP_ROWS)], sbuf.at[sid], sem_si
    )
    sp_in.start()

    ts_in.wait()
    ts_out = pltpu.make_async_copy(tbuf, out_hbm.at[pl.ds(base, _TS_ROWS)], sem_to)
    ts_out.start()
    sp_in.wait()
    sp_out = pltpu.make_async_copy(
        sbuf.at[sid], out_hbm.at[pl.ds(base + _TS_ROWS, _SP_ROWS)], sem_so
    )
    sp_out.start()

    ts_out.wait()
    sp_out.wait()


def kernel(x, W):
    del x
    return _pos_embed_copy(W)


# speedup vs baseline: 1.0492x; 1.0153x over previous
import functools

import jax
import jax.numpy as jnp
from jax import lax
from jax.experimental import pallas as pl
from jax.experimental.pallas import tpu as pltpu
from jax.experimental.pallas import tpu_sc as plsc

_MAX_LEN = 2048
_DIM = 1024
_NC = 2
_NS = 16
_NW = _NC * _NS
_ROWS_PER_W = _MAX_LEN // _NW
_TS_ROWS = 56
_SP_ROWS = _ROWS_PER_W - _TS_ROWS

_mesh = plsc.VectorSubcoreMesh(core_axis_name="c", subcore_axis_name="s")


@functools.partial(
    pl.kernel,
    mesh=_mesh,
    out_type=jax.ShapeDtypeStruct((_MAX_LEN, _DIM), jnp.float32),
    scratch_types=[
        pltpu.VMEM((_TS_ROWS, _DIM), jnp.float32),
        pltpu.VMEM_SHARED((_NS, _SP_ROWS, _DIM), jnp.float32),
        pltpu.SemaphoreType.DMA,
        pltpu.SemaphoreType.DMA,
        pltpu.SemaphoreType.DMA,
        pltpu.SemaphoreType.DMA,
    ],
)
def _pos_embed_copy(w_hbm, out_hbm, tbuf, sbuf, sem_ti, sem_to, sem_si, sem_so):
    sid = lax.axis_index("s")
    wid = sid * _NC + lax.axis_index("c")
    base = wid * _ROWS_PER_W

    ts_in = pltpu.make_async_copy(w_hbm.at[pl.ds(base, _TS_ROWS)], tbuf, sem_ti)
    ts_in.start()
    sp_in = pltpu.make_async_copy(
        w_hbm.at[pl.ds(base + _TS_ROWS, _SP_ROWS)], sbuf.at[sid], sem_si
    )
    sp_in.start()

    ts_in.wait()
    ts_out = pltpu.make_async_copy(tbuf, out_hbm.at[pl.ds(base, _TS_ROWS)], sem_to)
    ts_out.start()
    sp_in.wait()
    sp_out = pltpu.make_async_copy(
        sbuf.at[sid], out_hbm.at[pl.ds(base + _TS_ROWS, _SP_ROWS)], sem_so
    )
    sp_out.start()

    ts_out.wait()
    sp_out.wait()


def kernel(x, W):
    del x
    return _pos_embed_copy(W)
